# baseline (speedup 1.0000x reference)
import functools

import jax
import jax.numpy as jnp
from jax import lax
from jax.experimental import pallas as pl
from jax.experimental.pallas import tpu as pltpu
from jax.experimental.pallas import tpu_sc as plsc

_QP = 15
_COLS = 2048
_CHUNK_ROWS = 8


def _make_sc_call(rows, nc, ns, L, rows_per_w, rc, nchunks, dtype):
    mesh = plsc.VectorSubcoreMesh(
        core_axis_name="c", subcore_axis_name="s", num_cores=nc, num_subcores=ns
    )

    @functools.partial(
        pl.kernel,
        out_type=jax.ShapeDtypeStruct((rows, _COLS), dtype),
        mesh=mesh,
        compiler_params=pltpu.CompilerParams(
            needs_layout_passes=False, use_tc_tiling_on_sc=True
        ),
        scratch_types=[
            pltpu.VMEM((L,), jnp.float32),
            pltpu.VMEM((rc, _COLS), jnp.float32),
            pltpu.VMEM((rc, _COLS), jnp.float32),
            pltpu.VMEM((rc, _COLS), jnp.float32),
            pltpu.VMEM((rc, _COLS), jnp.float32),
            pltpu.SemaphoreType.DMA,
            pltpu.SemaphoreType.DMA,
            pltpu.SemaphoreType.DMA,
            pltpu.SemaphoreType.DMA,
        ],
    )
    def run(x_hbm, s_hbm, o_hbm, stab, ib0, ib1, ob0, ob1, si0, si1, so0, so1):
        wid = lax.axis_index("s") * nc + lax.axis_index("c")
        base = wid * rows_per_w

        def permute(tbl, idx):
            return jnp.take_along_axis(tbl, idx, axis=0, mode="promise_in_bounds")

        pltpu.sync_copy(s_hbm, stab)
        sv = stab[...]
        iota = lax.broadcasted_iota(jnp.int32, (L,), 0)
        cs = sv
        for d in (1, 2, 4, 8):
            g = permute(cs, jnp.maximum(iota - d, 0))
            cs = cs + jnp.where(iota >= d, g, jnp.float32(0.0))
        lv = permute(cs, jnp.maximum(iota - 1, 0))
        ltab_v = jnp.where(iota == 0, jnp.float32(0.0), lv)
        btab_v = cs - sv * 0.5
        b7v = permute(btab_v, jnp.full((L,), 7, jnp.int32))

        def compute(ibuf, obuf):
            @plsc.parallel_loop(0, (_COLS // 128) * rc, unroll=2)
            def inner(i):
                t = i >> 3
                r = i & (rc - 1)
                cbase = t * 128
                for s in range(128 // L):
                    xv = ibuf[r, pl.ds(cbase + s * L, L)]
                    idx = jnp.where(b7v < xv, 8, 0)
                    for step in (4, 2, 1):
                        bv = permute(btab_v, idx + (step - 1))
                        idx = idx + jnp.where(bv < xv, step, 0)
                    obuf[r, pl.ds(cbase + s * L, L)] = permute(ltab_v, idx)

        def in_slice(c):
            return x_hbm.at[pl.ds(base + c * rc, rc), :]

        def out_slice(c):
            return o_hbm.at[pl.ds(base + c * rc, rc), :]

        npairs = nchunks // 2
        pltpu.async_copy(in_slice(0), ib0, si0)

        def pair_body(p, carry):
            c0 = 2 * p
            pltpu.async_copy(in_slice(c0 + 1), ib1, si1)
            pltpu.make_async_copy(in_slice(c0), ib0, si0).wait()

            @pl.when(p > 0)
            def _():
                pltpu.make_async_copy(ob0, out_slice(c0), so0).wait()

            compute(ib0, ob0)
            pltpu.async_copy(ob0, out_slice(c0), so0)

            @pl.when(p + 1 < npairs)
            def _():
                pltpu.async_copy(in_slice(c0 + 2), ib0, si0)

            pltpu.make_async_copy(in_slice(c0 + 1), ib1, si1).wait()

            @pl.when(p > 0)
            def _():
                pltpu.make_async_copy(ob1, out_slice(c0 + 1), so1).wait()

            compute(ib1, ob1)
            pltpu.async_copy(ob1, out_slice(c0 + 1), so1)
            return carry

        lax.fori_loop(0, npairs, pair_body, 0)
        pltpu.make_async_copy(ob0, out_slice(nchunks - 2), so0).wait()
        pltpu.make_async_copy(ob1, out_slice(nchunks - 1), so1).wait()

    return run


def kernel(x, scale, Qn, Qp, num_elements, box_size):
    info = plsc.get_sparse_core_info()
    NC, NS, L = info.num_cores, info.num_subcores, info.num_lanes
    nw = NC * NS
    n = x.size
    rows = n // _COLS
    xf = x.reshape(rows, _COLS)
    scale16 = jnp.zeros((L,), x.dtype).at[: scale.shape[0]].set(scale)
    rows_per_w = rows // nw
    nchunks = rows_per_w // _CHUNK_ROWS
    run = _make_sc_call(rows, NC, NS, L, rows_per_w, _CHUNK_ROWS, nchunks, x.dtype)
    y = run(xf, scale16)
    return y.reshape(x.shape)


# speedup vs baseline: 1.1423x; 1.1423x over previous
import functools

import jax
import jax.numpy as jnp
from jax import lax
from jax.experimental import pallas as pl
from jax.experimental.pallas import tpu as pltpu
from jax.experimental.pallas import tpu_sc as plsc

_QP = 15
_COLS = 2048
_CHUNK_ROWS = 8
_SC_ROWS = 7168
_TC_BLOCK_ROWS = 256


def _make_sc_call(rows, sc_rows, nc, ns, L, rc, dtype):
    nw = nc * ns
    rows_per_w = sc_rows // nw
    nchunks = rows_per_w // rc
    mesh = plsc.VectorSubcoreMesh(
        core_axis_name="c", subcore_axis_name="s", num_cores=nc, num_subcores=ns
    )

    @functools.partial(
        pl.kernel,
        out_type=jax.ShapeDtypeStruct((sc_rows, _COLS), dtype),
        mesh=mesh,
        compiler_params=pltpu.CompilerParams(
            needs_layout_passes=False, use_tc_tiling_on_sc=True
        ),
        scratch_types=[
            pltpu.VMEM((L,), jnp.float32),
            pltpu.VMEM((rc, _COLS), jnp.float32),
            pltpu.VMEM((rc, _COLS), jnp.float32),
            pltpu.VMEM((rc, _COLS), jnp.float32),
            pltpu.VMEM((rc, _COLS), jnp.float32),
            pltpu.SemaphoreType.DMA,
            pltpu.SemaphoreType.DMA,
            pltpu.SemaphoreType.DMA,
            pltpu.SemaphoreType.DMA,
        ],
    )
    def run(x_hbm, s_hbm, o_hbm, stab, ib0, ib1, ob0, ob1, si0, si1, so0, so1):
        wid = lax.axis_index("s") * nc + lax.axis_index("c")
        base = wid * rows_per_w

        def permute(tbl, idx):
            return jnp.take_along_axis(tbl, idx, axis=0, mode="promise_in_bounds")

        pltpu.sync_copy(s_hbm, stab)
        sv = stab[...]
        iota = lax.broadcasted_iota(jnp.int32, (L,), 0)
        cs = sv
        for d in (1, 2, 4, 8):
            g = permute(cs, jnp.maximum(iota - d, 0))
            cs = cs + jnp.where(iota >= d, g, jnp.float32(0.0))
        lv = permute(cs, jnp.maximum(iota - 1, 0))
        ltab_v = jnp.where(iota == 0, jnp.float32(0.0), lv)
        btab_v = cs - sv * 0.5
        b7v = permute(btab_v, jnp.full((L,), 7, jnp.int32))

        def compute(ibuf, obuf):
            @plsc.parallel_loop(0, (_COLS // 128) * rc)
            def inner(i):
                t = i >> 3
                r = i & (rc - 1)
                cbase = t * 128
                for s in range(128 // L):
                    xv = ibuf[r, pl.ds(cbase + s * L, L)]
                    idx = jnp.where(b7v < xv, 8, 0)
                    for step in (4, 2, 1):
                        bv = permute(btab_v, idx + (step - 1))
                        idx = idx + jnp.where(bv < xv, step, 0)
                    obuf[r, pl.ds(cbase + s * L, L)] = permute(ltab_v, idx)

        def in_slice(c):
            return x_hbm.at[pl.ds(base + c * rc, rc), :]

        def out_slice(c):
            return o_hbm.at[pl.ds(base + c * rc, rc), :]

        npairs = nchunks // 2
        pltpu.async_copy(in_slice(0), ib0, si0)

        def pair_body(p, carry):
            c0 = 2 * p
            pltpu.async_copy(in_slice(c0 + 1), ib1, si1)
            pltpu.make_async_copy(in_slice(c0), ib0, si0).wait()

            @pl.when(p > 0)
            def _():
                pltpu.make_async_copy(ob0, out_slice(c0), so0).wait()

            compute(ib0, ob0)
            pltpu.async_copy(ob0, out_slice(c0), so0)

            @pl.when(p + 1 < npairs)
            def _():
                pltpu.async_copy(in_slice(c0 + 2), ib0, si0)

            pltpu.make_async_copy(in_slice(c0 + 1), ib1, si1).wait()

            @pl.when(p > 0)
            def _():
                pltpu.make_async_copy(ob1, out_slice(c0 + 1), so1).wait()

            compute(ib1, ob1)
            pltpu.async_copy(ob1, out_slice(c0 + 1), so1)
            return carry

        lax.fori_loop(0, npairs, pair_body, 0)
        pltpu.make_async_copy(ob0, out_slice(nchunks - 2), so0).wait()
        pltpu.make_async_copy(ob1, out_slice(nchunks - 1), so1).wait()

    return run


def _tc_body(scale_ref, x_ref, o_ref):
    x = x_ref[...]
    acc = jnp.zeros_like(x)
    c = x.dtype.type(0.0)
    for j in range(_QP):
        s = scale_ref[j]
        b = c + s * 0.5
        c = c + s
        acc = acc + jnp.where(x > b, s, jnp.zeros_like(s))
    o_ref[...] = acc


def kernel(x, scale, Qn, Qp, num_elements, box_size):
    info = plsc.get_sparse_core_info()
    NC, NS, L = info.num_cores, info.num_subcores, info.num_lanes
    n = x.size
    rows = n // _COLS
    xf = x.reshape(rows, _COLS)
    scale16 = jnp.zeros((L,), x.dtype).at[: scale.shape[0]].set(scale)

    run_sc = _make_sc_call(rows, _SC_ROWS, NC, NS, L, _CHUNK_ROWS, x.dtype)
    y_sc = run_sc(xf, scale16)

    tc_rows = rows - _SC_ROWS
    blk0 = _SC_ROWS // _TC_BLOCK_ROWS
    y_tc = pl.pallas_call(
        _tc_body,
        grid=(tc_rows // _TC_BLOCK_ROWS,),
        in_specs=[
            pl.BlockSpec(memory_space=pltpu.SMEM),
            pl.BlockSpec((_TC_BLOCK_ROWS, _COLS), lambda i: (i + blk0, 0)),
        ],
        out_specs=pl.BlockSpec((_TC_BLOCK_ROWS, _COLS), lambda i: (i, 0)),
        out_shape=jax.ShapeDtypeStruct((tc_rows, _COLS), x.dtype),
    )(scale, xf)

    return jnp.concatenate([y_sc, y_tc], axis=0).reshape(x.shape)
